# baseline (speedup 1.0000x reference)
import functools

import jax
import jax.numpy as jnp
import numpy as np
from jax import lax
from jax.experimental import pallas as pl
from jax.experimental.pallas import tpu as pltpu
from jax.experimental.pallas import tpu_sc as plsc

_LANES = 16
_NW = 32
_IDX_CHUNK = 128


def _make_gather(n_tok, vocab, dim, seq):
    per_w = n_tok // _NW
    n_chunks = per_w // _IDX_CHUNK
    mod = vocab - 1
    mesh = plsc.VectorSubcoreMesh(core_axis_name="c", subcore_axis_name="s")

    @functools.partial(
        pl.kernel,
        mesh=mesh,
        out_type=jax.ShapeDtypeStruct((n_tok, dim), jnp.float32),
        scratch_types=[
            pltpu.VMEM((per_w,), jnp.int32),
            pltpu.VMEM((per_w,), jnp.int32),
            pltpu.VMEM((n_chunks, _IDX_CHUNK), jnp.int32),
            pltpu.VMEM((per_w, dim), jnp.float32),
            pltpu.SemaphoreType.DMA,
        ],
        compiler_params=pltpu.CompilerParams(use_tc_tiling_on_sc=False),
    )
    def gather_kernel(tok_hbm, tokp_hbm, table_hbm, h_hbm,
                      cur_v, prev_v, idx_v, rows_v, sem):
        wid = lax.axis_index("s") * 2 + lax.axis_index("c")
        base = wid * per_w
        pltpu.sync_copy(tok_hbm.at[pl.ds(base, per_w)], cur_v)
        pltpu.sync_copy(tokp_hbm.at[pl.ds(base, per_w)], prev_v)
        modv = jnp.full((_LANES,), mod, dtype=jnp.int32)
        for i in range(per_w // _LANES):
            cur = cur_v[pl.ds(i * _LANES, _LANES)]
            prev = prev_v[pl.ds(i * _LANES, _LANES)]
            h = (cur * 36313) ^ (prev * 27191)
            h = lax.rem(h, modv)
            pos = base + i * _LANES + lax.iota(jnp.int32, _LANES)
            h = jnp.where((pos & (seq - 1)) == 0, mod, h)
            idx_v[(i * _LANES) // _IDX_CHUNK,
                  pl.ds((i * _LANES) % _IDX_CHUNK, _LANES)] = h
        copies = []
        for j in range(n_chunks):
            copies.append(pltpu.async_copy(
                table_hbm.at[idx_v.at[jnp.int32(j)]],
                rows_v.at[pl.ds(j * _IDX_CHUNK, _IDX_CHUNK)],
                sem))
        for c in copies:
            c.wait()
        pltpu.sync_copy(rows_v, h_hbm.at[pl.ds(base, per_w)])

    return gather_kernel


_ZERO = np.int32(0)


def _mm_body(h_ref, wt_ref, scale_ref, o_ref):
    o_ref[...] = jnp.dot(h_ref[...], wt_ref[...],
                         preferred_element_type=jnp.float32) * scale_ref[0]


def _make_matmul(n_tok, dim, model_dim, block_rows=1024):
    grid = n_tok // block_rows
    return pl.pallas_call(
        _mm_body,
        grid=(grid,),
        in_specs=[
            pl.BlockSpec((block_rows, dim), lambda i: (i, _ZERO)),
            pl.BlockSpec((dim, model_dim), lambda i: (_ZERO, _ZERO)),
            pl.BlockSpec((1,), lambda i: (_ZERO,), memory_space=pltpu.SMEM),
        ],
        out_specs=pl.BlockSpec((block_rows, model_dim), lambda i: (i, _ZERO)),
        out_shape=jax.ShapeDtypeStruct((n_tok, model_dim), jnp.float32),
    )


def kernel(token_ids, table, W_proj, scale):
    batch, seq = token_ids.shape
    vocab, dim = table.shape
    model_dim = W_proj.shape[0]
    n_tok = batch * seq

    tok = token_ids.astype(jnp.int32).reshape(-1)
    tok_prev = jnp.concatenate([jnp.zeros((1,), jnp.int32), tok[:-1]])

    h = _make_gather(n_tok, vocab, dim, seq)(tok, tok_prev, table)
    wt = W_proj.T.astype(jnp.float32)
    scale1 = jnp.reshape(scale, (1,)).astype(jnp.float32)
    out = _make_matmul(n_tok, dim, model_dim)(h, wt, scale1)
    return out.reshape(batch, seq, model_dim)
